# baseline (speedup 1.0000x reference)
import jax
import jax.numpy as jnp
from jax.experimental import pallas as pl
from jax.experimental.pallas import tpu as pltpu

B = 2
DIM = 192
HEADS = 4
CH = DIM // HEADS
H = 32
W = 32
N = H * W
KKS = (int(N / 2), int(N * 2 / 3), int(N * 3 / 4), int(N * 4 / 5))
INT_MIN = -2147483648
MASK31 = 0x7FFFFFFF


def _rne_bf16(x):
    u = jax.lax.bitcast_convert_type(x, jnp.uint32)
    r = u + jnp.uint32(0x7FFF) + ((u >> 16) & jnp.uint32(1))
    r = r & jnp.uint32(0xFFFF0000)
    return jax.lax.bitcast_convert_type(r, jnp.float32)


def _depthwise3x3(x, w):
    p = jnp.pad(x, ((0, 0), (0, 0), (1, 1), (1, 1)))
    acc = None
    for dy in range(3):
        for dx in range(3):
            t = (p[:, :, dy:dy + H, dx:dx + W]
                 * w[None, :, 0, dy, dx, None, None])
            acc = t if acc is None else acc + t
    return acc


def _conv2d(x, w, stride=1, padding=0, dilation=1, groups=1):
    return jax.lax.conv_general_dilated(
        x, w, (stride, stride), ((padding, padding), (padding, padding)),
        rhs_dilation=(dilation, dilation),
        dimension_numbers=('NCHW', 'OIHW', 'NCHW'),
        feature_group_count=groups)


def _attn_core_kernel(temp_ref, avec_ref, q_ref, k_ref, v_ref, pow_ref,
                      o_ref):
    i = pl.program_id(0)
    q = q_ref[0]
    k = k_ref[0]
    v = v_ref[0]

    qn = q / jnp.maximum(
        jnp.sqrt(jnp.sum(q * q, axis=1, keepdims=True)), 1e-12)
    kn = k / jnp.maximum(
        jnp.sqrt(jnp.sum(k * k, axis=1, keepdims=True)), 1e-12)

    attnT = jax.lax.dot_general(
        kn, qn, (((0,), (0,)), ((), ())),
        preferred_element_type=jnp.float32) * temp_ref[i]

    colmax = jnp.max(attnT, axis=0, keepdims=True)
    e = jnp.exp(attnT - colmax)

    bits = jax.lax.bitcast_convert_type(attnT, jnp.int32)
    key = jnp.where(bits >= 0, bits, bits ^ MASK31)

    def kth_largest(kk):
        cnt0 = jnp.sum((key >= 0).astype(jnp.int32), axis=0, keepdims=True)
        prefix = jnp.where(cnt0 >= kk, jnp.int32(0), jnp.int32(INT_MIN))

        def body(j, prefix):
            bit = jax.lax.shift_left(jnp.int32(1), 30 - j)
            trial = prefix | bit
            cnt = jnp.sum((key >= trial).astype(jnp.int32), axis=0,
                          keepdims=True)
            return jnp.where(cnt >= kk, trial, prefix)

        prefix = jax.lax.fori_loop(0, 31, body, prefix)
        fb = jnp.where(prefix >= 0, prefix, prefix ^ MASK31)
        return jax.lax.bitcast_convert_type(fb, jnp.float32)

    f = jnp.zeros_like(attnT)
    for idx in range(4):
        t = kth_largest(KKS[idx])
        m = attnT >= t
        s = jnp.sum(jnp.where(m, e, 0.0), axis=0, keepdims=True)
        f = f + jnp.where(m, avec_ref[idx] / s, 0.0)
    wmat = e * f

    out_h = jax.lax.dot_general(
        v, wmat, (((1,), (0,)), ((), ())),
        preferred_element_type=jnp.float32)

    contrib = jax.lax.dot_general(
        pow_ref[0], out_h, (((1,), (0,)), ((), ())),
        preferred_element_type=jnp.float32)

    @pl.when(i % HEADS == 0)
    def _():
        o_ref[0] = contrib

    @pl.when(i % HEADS != 0)
    def _():
        o_ref[0] += contrib


def _attn_core(q, k, v, temp_full, avec, po_w):
    return pl.pallas_call(
        _attn_core_kernel,
        grid=(B * HEADS,),
        in_specs=[
            pl.BlockSpec(memory_space=pltpu.SMEM),
            pl.BlockSpec(memory_space=pltpu.SMEM),
            pl.BlockSpec((1, CH, N), lambda i: (i, 0, 0)),
            pl.BlockSpec((1, CH, N), lambda i: (i, 0, 0)),
            pl.BlockSpec((1, CH, N), lambda i: (i, 0, 0)),
            pl.BlockSpec((1, DIM, CH), lambda i: (i % HEADS, 0, 0)),
        ],
        out_specs=pl.BlockSpec((1, DIM, N), lambda i: (i // HEADS, 0, 0)),
        out_shape=jax.ShapeDtypeStruct((B, DIM, N), jnp.float32),
    )(temp_full, avec, q, k, v, po_w)


def kernel(x, pe_w, pe_b, ln_g, ln_b, aspp1_w, bn1_g, bn1_b, aspp2_w, bn2_g,
           bn2_b, asppp_w, bnp_g, bnp_b, kv_w, kvdw_w, po_w, temperature,
           a1, a2, a3, a4):
    b, c, h, w = x.shape
    heads = HEADS
    ch = c // heads

    pe = _conv2d(x, pe_w) + pe_b[None, :, None, None]
    pe = pe.transpose(0, 2, 3, 1)
    mu = pe.mean(-1, keepdims=True)
    var = pe.var(-1, keepdims=True)
    pe = (pe - mu) / jnp.sqrt(var + 1e-5) * ln_g + ln_b
    pe = pe.transpose(0, 3, 1, 2)
    x = x + pe

    def bn(y, g, be):
        return (y / jnp.sqrt(1.0 + 1e-5) * g[None, :, None, None]
                + be[None, :, None, None])

    q1 = jax.nn.relu(bn(_conv2d(x, aspp1_w, padding=3, dilation=3),
                        bn1_g, bn1_b))
    q2 = jax.nn.relu(bn(_conv2d(x, aspp2_w, padding=5, dilation=5),
                        bn2_g, bn2_b))
    q = jax.nn.relu(bn(_conv2d(jnp.concatenate([q1, q2], axis=1), asppp_w),
                       bnp_g, bnp_b))

    kv = _depthwise3x3(_rne_bf16(_conv2d(x, kv_w)), kvdw_w)
    k, v = jnp.split(kv, 2, axis=1)

    nn = h * w
    q = q.reshape(b * heads, ch, nn)
    k = k.reshape(b * heads, ch, nn)
    v = v.reshape(b * heads, ch, nn)

    temp_full = jnp.tile(temperature.reshape(heads), (b,))
    avec = jnp.concatenate([a1, a2, a3, a4]).astype(jnp.float32)

    out = _attn_core(q, k, v, temp_full, avec,
                     po_w.reshape(c, heads, ch).transpose(1, 0, 2))
    return out.reshape(b, c, h, w)


# speedup vs baseline: 1.0075x; 1.0075x over previous
import jax
import jax.numpy as jnp
from jax.experimental import pallas as pl
from jax.experimental.pallas import tpu as pltpu

B = 2
DIM = 192
HEADS = 4
CH = DIM // HEADS
H = 32
W = 32
N = H * W
KKS = (int(N / 2), int(N * 2 / 3), int(N * 3 / 4), int(N * 4 / 5))
INT_MIN = -2147483648
MASK31 = 0x7FFFFFFF


def _rne_bf16(x):
    u = jax.lax.bitcast_convert_type(x, jnp.uint32)
    r = u + jnp.uint32(0x7FFF) + ((u >> 16) & jnp.uint32(1))
    r = r & jnp.uint32(0xFFFF0000)
    return jax.lax.bitcast_convert_type(r, jnp.float32)


def _depthwise3x3(x, w):
    p = jnp.pad(x, ((0, 0), (0, 0), (1, 1), (1, 1)))
    acc = None
    for dy in range(3):
        for dx in range(3):
            t = (p[:, :, dy:dy + H, dx:dx + W]
                 * w[None, :, 0, dy, dx, None, None])
            acc = t if acc is None else acc + t
    return acc


def _conv2d(x, w, stride=1, padding=0, dilation=1, groups=1):
    return jax.lax.conv_general_dilated(
        x, w, (stride, stride), ((padding, padding), (padding, padding)),
        rhs_dilation=(dilation, dilation),
        dimension_numbers=('NCHW', 'OIHW', 'NCHW'),
        feature_group_count=groups)


def _rne_bf16_i32(x):
    u = jax.lax.bitcast_convert_type(x, jnp.int32)
    r = u + jnp.int32(0x7FFF) + ((u >> 16) & jnp.int32(1))
    r = r & jnp.int32(-65536)
    return jax.lax.bitcast_convert_type(r, jnp.float32)


def _dw_in_kernel(xr, wtaps):
    zp = jnp.zeros((CH, 64), jnp.float32)
    xp = jnp.concatenate([zp, xr, zp], axis=1)
    lane = jax.lax.broadcasted_iota(jnp.int32, (1, N), 1)
    row = lane // W
    col = lane % W
    acc = None
    for dy in range(3):
        for dx in range(3):
            s = (dy - 1) * W + (dx - 1)
            sl = jax.lax.slice(xp, (0, 64 + s), (CH, 64 + s + N))
            ok = ((row + dy - 1 >= 0) & (row + dy - 1 < H)
                  & (col + dx - 1 >= 0) & (col + dx - 1 < W))
            t = jnp.where(ok, sl * wtaps[:, 3 * dy + dx][:, None], 0.0)
            acc = t if acc is None else acc + t
    return acc


def _attn_core_kernel(temp_ref, avec_ref, q_ref, kv1k_ref, kv1v_ref,
                      wk_ref, wv_ref, pow_ref, o_ref):
    i = pl.program_id(0)
    q = q_ref[0]
    k = _dw_in_kernel(_rne_bf16_i32(kv1k_ref[0, 0]), wk_ref[0])
    v = _dw_in_kernel(_rne_bf16_i32(kv1v_ref[0, 0]), wv_ref[0])

    qn = q / jnp.maximum(
        jnp.sqrt(jnp.sum(q * q, axis=1, keepdims=True)), 1e-12)
    kn = k / jnp.maximum(
        jnp.sqrt(jnp.sum(k * k, axis=1, keepdims=True)), 1e-12)

    attnT = jax.lax.dot_general(
        kn, qn, (((0,), (0,)), ((), ())),
        preferred_element_type=jnp.float32) * temp_ref[i]

    colmax = jnp.max(attnT, axis=0, keepdims=True)
    e = jnp.exp(attnT - colmax)

    bits = jax.lax.bitcast_convert_type(attnT, jnp.int32)
    key = jnp.where(bits >= 0, bits, bits ^ MASK31)

    def kth_largest(kk):
        cnt0 = jnp.sum((key >= 0).astype(jnp.int32), axis=0, keepdims=True)
        prefix = jnp.where(cnt0 >= kk, jnp.int32(0), jnp.int32(INT_MIN))

        def body(j, prefix):
            bit = jax.lax.shift_left(jnp.int32(1), 30 - j)
            trial = prefix | bit
            cnt = jnp.sum((key >= trial).astype(jnp.int32), axis=0,
                          keepdims=True)
            return jnp.where(cnt >= kk, trial, prefix)

        prefix = jax.lax.fori_loop(0, 31, body, prefix)
        fb = jnp.where(prefix >= 0, prefix, prefix ^ MASK31)
        return jax.lax.bitcast_convert_type(fb, jnp.float32)

    f = jnp.zeros_like(attnT)
    for idx in range(4):
        t = kth_largest(KKS[idx])
        m = attnT >= t
        s = jnp.sum(jnp.where(m, e, 0.0), axis=0, keepdims=True)
        f = f + jnp.where(m, avec_ref[idx] / s, 0.0)
    wmat = e * f

    out_h = jax.lax.dot_general(
        v, wmat, (((1,), (0,)), ((), ())),
        preferred_element_type=jnp.float32)

    contrib = jax.lax.dot_general(
        pow_ref[0], out_h, (((1,), (0,)), ((), ())),
        preferred_element_type=jnp.float32)

    @pl.when(i % HEADS == 0)
    def _():
        o_ref[0] = contrib

    @pl.when(i % HEADS != 0)
    def _():
        o_ref[0] += contrib


def _attn_core(q, kv1, temp_full, avec, dw_w, po_w):
    return pl.pallas_call(
        _attn_core_kernel,
        grid=(B * HEADS,),
        in_specs=[
            pl.BlockSpec(memory_space=pltpu.SMEM),
            pl.BlockSpec(memory_space=pltpu.SMEM),
            pl.BlockSpec((1, CH, N), lambda i: (i, 0, 0)),
            pl.BlockSpec((1, 1, CH, N), lambda i: (i // HEADS, i % HEADS, 0, 0)),
            pl.BlockSpec((1, 1, CH, N),
                         lambda i: (i // HEADS, HEADS + i % HEADS, 0, 0)),
            pl.BlockSpec((1, CH, 9), lambda i: (i % HEADS, 0, 0)),
            pl.BlockSpec((1, CH, 9), lambda i: (HEADS + i % HEADS, 0, 0)),
            pl.BlockSpec((1, DIM, CH), lambda i: (i % HEADS, 0, 0)),
        ],
        out_specs=pl.BlockSpec((1, DIM, N), lambda i: (i // HEADS, 0, 0)),
        out_shape=jax.ShapeDtypeStruct((B, DIM, N), jnp.float32),
    )(temp_full, avec, q, kv1, kv1, dw_w, dw_w, po_w)


def kernel(x, pe_w, pe_b, ln_g, ln_b, aspp1_w, bn1_g, bn1_b, aspp2_w, bn2_g,
           bn2_b, asppp_w, bnp_g, bnp_b, kv_w, kvdw_w, po_w, temperature,
           a1, a2, a3, a4):
    b, c, h, w = x.shape
    heads = HEADS
    ch = c // heads

    pe = _conv2d(x, pe_w) + pe_b[None, :, None, None]
    pe = pe.transpose(0, 2, 3, 1)
    mu = pe.mean(-1, keepdims=True)
    var = pe.var(-1, keepdims=True)
    pe = (pe - mu) / jnp.sqrt(var + 1e-5) * ln_g + ln_b
    pe = pe.transpose(0, 3, 1, 2)
    x = x + pe

    def bn(y, g, be):
        return (y / jnp.sqrt(1.0 + 1e-5) * g[None, :, None, None]
                + be[None, :, None, None])

    q1 = jax.nn.relu(bn(_conv2d(x, aspp1_w, padding=3, dilation=3),
                        bn1_g, bn1_b))
    q2 = jax.nn.relu(bn(_conv2d(x, aspp2_w, padding=5, dilation=5),
                        bn2_g, bn2_b))
    q = jax.nn.relu(bn(_conv2d(jnp.concatenate([q1, q2], axis=1), asppp_w),
                       bnp_g, bnp_b))

    nn = h * w
    kv1 = _conv2d(x, kv_w).reshape(b, 2 * heads, ch, nn)
    dw_w = kvdw_w.reshape(2 * heads, ch, 9)

    q = q.reshape(b * heads, ch, nn)

    temp_full = jnp.tile(temperature.reshape(heads), (b,))
    avec = jnp.concatenate([a1, a2, a3, a4]).astype(jnp.float32)

    out = _attn_core(q, kv1, temp_full, avec, dw_w,
                     po_w.reshape(c, heads, ch).transpose(1, 0, 2))
    return out.reshape(b, c, h, w)


# speedup vs baseline: 1.0078x; 1.0003x over previous
import jax
import jax.numpy as jnp
from jax.experimental import pallas as pl
from jax.experimental.pallas import tpu as pltpu

B = 2
DIM = 192
HEADS = 4
CH = DIM // HEADS
H = 32
W = 32
N = H * W
KKS = (int(N / 2), int(N * 2 / 3), int(N * 3 / 4), int(N * 4 / 5))
INT_MIN = -2147483648
MASK31 = 0x7FFFFFFF


def _conv2d(x, w, stride=1, padding=0, dilation=1, groups=1):
    return jax.lax.conv_general_dilated(
        x, w, (stride, stride), ((padding, padding), (padding, padding)),
        rhs_dilation=(dilation, dilation),
        dimension_numbers=('NCHW', 'OIHW', 'NCHW'),
        feature_group_count=groups)


def _rne_bf16_i32(x):
    u = jax.lax.bitcast_convert_type(x, jnp.int32)
    r = u + jnp.int32(0x7FFF) + ((u >> 16) & jnp.int32(1))
    r = r & jnp.int32(-65536)
    return jax.lax.bitcast_convert_type(r, jnp.float32)


def _dw_in_kernel(xr, wtaps):
    zp = jnp.zeros((CH, 64), jnp.float32)
    xp = jnp.concatenate([zp, xr, zp], axis=1)
    lane = jax.lax.broadcasted_iota(jnp.int32, (1, N), 1)
    row = lane // W
    col = lane % W
    acc = None
    for dy in range(3):
        for dx in range(3):
            s = (dy - 1) * W + (dx - 1)
            sl = jax.lax.slice(xp, (0, 64 + s), (CH, 64 + s + N))
            ok = ((row + dy - 1 >= 0) & (row + dy - 1 < H)
                  & (col + dx - 1 >= 0) & (col + dx - 1 < W))
            t = jnp.where(ok, sl * wtaps[:, 3 * dy + dx][:, None], 0.0)
            acc = t if acc is None else acc + t
    return acc


def _attn_core_kernel(temp_ref, avec_ref, q_ref, kv1k_ref, kv1v_ref,
                      wk_ref, wv_ref, pow_ref, o_ref):
    i = pl.program_id(0)
    q = q_ref[0]
    k = _dw_in_kernel(_rne_bf16_i32(kv1k_ref[0, 0]), wk_ref[0])
    v = _dw_in_kernel(_rne_bf16_i32(kv1v_ref[0, 0]), wv_ref[0])

    qn = q / jnp.maximum(
        jnp.sqrt(jnp.sum(q * q, axis=1, keepdims=True)), 1e-12)
    kn = k / jnp.maximum(
        jnp.sqrt(jnp.sum(k * k, axis=1, keepdims=True)), 1e-12)

    attnT = jax.lax.dot_general(
        kn, qn, (((0,), (0,)), ((), ())),
        preferred_element_type=jnp.float32) * temp_ref[i]

    colmax = jnp.max(attnT, axis=0, keepdims=True)
    e = jnp.exp(attnT - colmax)

    bits = jax.lax.bitcast_convert_type(attnT, jnp.int32)
    key = jnp.where(bits >= 0, bits, bits ^ MASK31)

    def kth_largest(kk):
        cnt0 = jnp.sum((key >= 0).astype(jnp.int32), axis=0, keepdims=True)
        prefix = jnp.where(cnt0 >= kk, jnp.int32(0), jnp.int32(INT_MIN))

        def body(j, prefix):
            bit = jax.lax.shift_left(jnp.int32(1), 30 - j)
            trial = prefix | bit
            cnt = jnp.sum((key >= trial).astype(jnp.int32), axis=0,
                          keepdims=True)
            return jnp.where(cnt >= kk, trial, prefix)

        prefix = jax.lax.fori_loop(0, 31, body, prefix)
        fb = jnp.where(prefix >= 0, prefix, prefix ^ MASK31)
        return jax.lax.bitcast_convert_type(fb, jnp.float32)

    f = jnp.zeros_like(attnT)
    for idx in range(4):
        t = kth_largest(KKS[idx])
        m = attnT >= t
        s = jnp.sum(jnp.where(m, e, 0.0), axis=0, keepdims=True)
        f = f + jnp.where(m, avec_ref[idx] / s, 0.0)
    wmat = e * f

    out_h = jax.lax.dot_general(
        v, wmat, (((1,), (0,)), ((), ())),
        preferred_element_type=jnp.float32)

    contrib = jax.lax.dot_general(
        pow_ref[0], out_h, (((1,), (0,)), ((), ())),
        preferred_element_type=jnp.float32)

    @pl.when(i % HEADS == 0)
    def _():
        o_ref[0] = contrib

    @pl.when(i % HEADS != 0)
    def _():
        o_ref[0] += contrib


def _attn_core(q, kv1, temp_full, avec, dw_w, po_w):
    return pl.pallas_call(
        _attn_core_kernel,
        grid=(B * HEADS,),
        in_specs=[
            pl.BlockSpec(memory_space=pltpu.SMEM),
            pl.BlockSpec(memory_space=pltpu.SMEM),
            pl.BlockSpec((1, CH, N), lambda i: (i, 0, 0)),
            pl.BlockSpec((1, 1, CH, N), lambda i: (i // HEADS, i % HEADS, 0, 0)),
            pl.BlockSpec((1, 1, CH, N),
                         lambda i: (i // HEADS, HEADS + i % HEADS, 0, 0)),
            pl.BlockSpec((1, CH, 9), lambda i: (i % HEADS, 0, 0)),
            pl.BlockSpec((1, CH, 9), lambda i: (HEADS + i % HEADS, 0, 0)),
            pl.BlockSpec((1, DIM, CH), lambda i: (i % HEADS, 0, 0)),
        ],
        out_specs=pl.BlockSpec((1, DIM, N), lambda i: (i // HEADS, 0, 0)),
        out_shape=jax.ShapeDtypeStruct((B, DIM, N), jnp.float32),
    )(temp_full, avec, q, kv1, kv1, dw_w, dw_w, po_w)


def kernel(x, pe_w, pe_b, ln_g, ln_b, aspp1_w, bn1_g, bn1_b, aspp2_w, bn2_g,
           bn2_b, asppp_w, bnp_g, bnp_b, kv_w, kvdw_w, po_w, temperature,
           a1, a2, a3, a4):
    b, c, h, w = x.shape
    heads = HEADS
    ch = c // heads

    pe = _conv2d(x, pe_w) + pe_b[None, :, None, None]
    pe = pe.transpose(0, 2, 3, 1)
    mu = pe.mean(-1, keepdims=True)
    var = pe.var(-1, keepdims=True)
    pe = (pe - mu) / jnp.sqrt(var + 1e-5) * ln_g + ln_b
    pe = pe.transpose(0, 3, 1, 2)
    x = x + pe

    def bn(y, g, be):
        return (y / jnp.sqrt(1.0 + 1e-5) * g[None, :, None, None]
                + be[None, :, None, None])

    q1 = jax.nn.relu(bn(_conv2d(x, aspp1_w, padding=3, dilation=3),
                        bn1_g, bn1_b))
    q2 = jax.nn.relu(bn(_conv2d(x, aspp2_w, padding=5, dilation=5),
                        bn2_g, bn2_b))
    q = jax.nn.relu(bn(_conv2d(jnp.concatenate([q1, q2], axis=1), asppp_w),
                       bnp_g, bnp_b))

    nn = h * w
    kv1 = _conv2d(x, kv_w).reshape(b, 2 * heads, ch, nn)
    dw_w = kvdw_w.reshape(2 * heads, ch, 9)

    q = q.reshape(b * heads, ch, nn)

    temp_full = jnp.tile(temperature.reshape(heads), (b,))
    avec = jnp.concatenate([a1, a2, a3, a4]).astype(jnp.float32)

    out = _attn_core(q, kv1, temp_full, avec, dw_w,
                     po_w.reshape(c, heads, ch).transpose(1, 0, 2))
    return out.reshape(b, c, h, w)
